# baseline (speedup 1.0000x reference)
import jax
import jax.numpy as jnp
from jax import lax
from jax.experimental import pallas as pl
from jax.experimental.pallas import tpu as pltpu

N, D, H = 10000, 128, 128
SIZES = (1024, 4096, 4096, 784)
OFFS = (0, 1024, 5120, 9216)
K = len(SIZES)


def _gclstm_kernel(x_hbm, h_hbm, c_hbm,
                   wi_ref, thi_ref, bi_ref,
                   wf_ref, thf_ref, bf_ref,
                   wc_ref, thc_ref, bc_ref,
                   wo_ref, tho_ref, bo_ref,
                   wlin_ref, blin_ref,
                   hout_hbm, h0_hbm, cnew_hbm,
                   x_v, h_v, c_v, ho_v, h0_v, cn_v,
                   in_sem, out_sem):
    ins = []
    for k in range(K):
        sl = pl.ds(OFFS[k], SIZES[k])
        cps = []
        for j, (src, dst) in enumerate(
                ((x_hbm, x_v), (h_hbm, h_v), (c_hbm, c_v))):
            cp = pltpu.make_async_copy(src.at[sl, :], dst.at[sl, :],
                                       in_sem.at[k, j])
            cp.start()
            cps.append(cp)
        ins.append(cps)

    out_cps = []
    for k in range(K):
        sl = pl.ds(OFFS[k], SIZES[k])
        for cp in ins[k]:
            cp.wait()
        x = x_v[sl, :]
        hh = h_v[sl, :]

        def gate(w_ref, th_ref, b_ref):
            g = jnp.dot(x, w_ref[...], preferred_element_type=jnp.float32)
            g = g + jnp.dot(hh, th_ref[...],
                            preferred_element_type=jnp.float32)
            return g + b_ref[...]

        i = jax.nn.sigmoid(gate(wi_ref, thi_ref, bi_ref))
        f = jax.nn.sigmoid(gate(wf_ref, thf_ref, bf_ref))
        t = jnp.tanh(gate(wc_ref, thc_ref, bc_ref))
        o = jax.nn.sigmoid(gate(wo_ref, tho_ref, bo_ref))
        cn = f * c_v[sl, :] + i * t
        h0 = o * jnp.tanh(cn)
        cn_v[sl, :] = cn
        h0_v[sl, :] = h0
        ho_v[sl, :] = lax.dot_general(
            jnp.maximum(h0, 0.0), wlin_ref[...],
            dimension_numbers=(((1,), (1,)), ((), ())),
            preferred_element_type=jnp.float32) + blin_ref[...]
        for j, (src, dst) in enumerate(
                ((ho_v, hout_hbm), (h0_v, h0_hbm), (cn_v, cnew_hbm))):
            cp = pltpu.make_async_copy(src.at[sl, :], dst.at[sl, :],
                                       out_sem.at[k, j])
            cp.start()
            out_cps.append(cp)

    for cp in out_cps:
        cp.wait()


def kernel(x, edge_index, edge_weight, h, c,
           W_i, b_i, Theta_i, bc_i,
           W_f, b_f, Theta_f, bc_f,
           W_c, b_c, Theta_c, bc_c,
           W_o, b_o, Theta_o, bc_o,
           W_lin, b_lin):
    del edge_index, edge_weight
    bias_i = b_i + bc_i.reshape(1, H)
    bias_f = b_f + bc_f.reshape(1, H)
    bias_c = b_c + bc_c.reshape(1, H)
    bias_o = b_o + bc_o.reshape(1, H)
    blin = b_lin.reshape(1, H)

    any_spec = pl.BlockSpec(memory_space=pl.ANY)
    wspec = pl.BlockSpec((D, H), lambda: (0, 0))
    bspec = pl.BlockSpec((1, H), lambda: (0, 0))
    vmem = lambda: pltpu.VMEM((N, H), jnp.float32)

    hout, h0, cnew = pl.pallas_call(
        _gclstm_kernel,
        in_specs=[
            any_spec, any_spec, any_spec,
            wspec, wspec, bspec,
            wspec, wspec, bspec,
            wspec, wspec, bspec,
            wspec, wspec, bspec,
            wspec, bspec,
        ],
        out_specs=[any_spec, any_spec, any_spec],
        out_shape=[
            jax.ShapeDtypeStruct((N, H), jnp.float32),
            jax.ShapeDtypeStruct((N, H), jnp.float32),
            jax.ShapeDtypeStruct((N, H), jnp.float32),
        ],
        scratch_shapes=[
            vmem(), vmem(), vmem(),
            vmem(), vmem(), vmem(),
            pltpu.SemaphoreType.DMA((K, 3)),
            pltpu.SemaphoreType.DMA((K, 3)),
        ],
    )(x, h, c,
      W_i, Theta_i, bias_i,
      W_f, Theta_f, bias_f,
      W_c, Theta_c, bias_c,
      W_o, Theta_o, bias_o,
      W_lin, blin)
    return (hout, h0, cnew)


# speedup vs baseline: 1.0013x; 1.0013x over previous
import jax
import jax.numpy as jnp
from jax import lax
from jax.experimental import pallas as pl
from jax.experimental.pallas import tpu as pltpu

N, D, H = 10000, 128, 128
SIZES = (1024, 4096, 4096, 784)
OFFS = (0, 1024, 5120, 9216)
K = len(SIZES)


def _gclstm_kernel(x_hbm, h_hbm, c_hbm,
                   wi_ref, thi_ref, bi_ref,
                   wf_ref, thf_ref, bf_ref,
                   wc_ref, thc_ref, bc_ref,
                   wo_ref, tho_ref, bo_ref,
                   wlin_ref, blin_ref,
                   hout_hbm, h0_hbm, cnew_hbm,
                   x_v, h_v, c_v, ho_v, h0_v, cn_v,
                   in_sem, out_sem):
    def start_ins(k):
        sl = pl.ds(OFFS[k], SIZES[k])
        cps = []
        for j, (src, dst) in enumerate(
                ((x_hbm, x_v), (h_hbm, h_v), (c_hbm, c_v))):
            cp = pltpu.make_async_copy(src.at[sl, :], dst.at[sl, :],
                                       in_sem.at[k, j])
            cp.start()
            cps.append(cp)
        return cps

    LOOKAHEAD = 2
    ins = [start_ins(k) for k in range(min(LOOKAHEAD, K))]

    out_cps = []
    for k in range(K):
        if k + LOOKAHEAD < K:
            ins.append(start_ins(k + LOOKAHEAD))
        sl = pl.ds(OFFS[k], SIZES[k])
        for cp in ins[k]:
            cp.wait()
        x = x_v[sl, :]
        hh = h_v[sl, :]

        def gate(w_ref, th_ref, b_ref):
            g = jnp.dot(x, w_ref[...], preferred_element_type=jnp.float32)
            g = g + jnp.dot(hh, th_ref[...],
                            preferred_element_type=jnp.float32)
            return g + b_ref[...]

        i = jax.nn.sigmoid(gate(wi_ref, thi_ref, bi_ref))
        f = jax.nn.sigmoid(gate(wf_ref, thf_ref, bf_ref))
        t = jnp.tanh(gate(wc_ref, thc_ref, bc_ref))
        o = jax.nn.sigmoid(gate(wo_ref, tho_ref, bo_ref))
        cn = f * c_v[sl, :] + i * t
        h0 = o * jnp.tanh(cn)
        cn_v[sl, :] = cn
        h0_v[sl, :] = h0
        ho_v[sl, :] = lax.dot_general(
            jnp.maximum(h0, 0.0), wlin_ref[...],
            dimension_numbers=(((1,), (1,)), ((), ())),
            preferred_element_type=jnp.float32) + blin_ref[...]
        for j, (src, dst) in enumerate(
                ((ho_v, hout_hbm), (h0_v, h0_hbm), (cn_v, cnew_hbm))):
            cp = pltpu.make_async_copy(src.at[sl, :], dst.at[sl, :],
                                       out_sem.at[k, j])
            cp.start()
            out_cps.append(cp)

    for cp in out_cps:
        cp.wait()


def kernel(x, edge_index, edge_weight, h, c,
           W_i, b_i, Theta_i, bc_i,
           W_f, b_f, Theta_f, bc_f,
           W_c, b_c, Theta_c, bc_c,
           W_o, b_o, Theta_o, bc_o,
           W_lin, b_lin):
    del edge_index, edge_weight
    bias_i = b_i + bc_i.reshape(1, H)
    bias_f = b_f + bc_f.reshape(1, H)
    bias_c = b_c + bc_c.reshape(1, H)
    bias_o = b_o + bc_o.reshape(1, H)
    blin = b_lin.reshape(1, H)

    any_spec = pl.BlockSpec(memory_space=pl.ANY)
    wspec = pl.BlockSpec((D, H), lambda: (0, 0))
    bspec = pl.BlockSpec((1, H), lambda: (0, 0))
    vmem = lambda: pltpu.VMEM((N, H), jnp.float32)

    hout, h0, cnew = pl.pallas_call(
        _gclstm_kernel,
        in_specs=[
            any_spec, any_spec, any_spec,
            wspec, wspec, bspec,
            wspec, wspec, bspec,
            wspec, wspec, bspec,
            wspec, wspec, bspec,
            wspec, bspec,
        ],
        out_specs=[any_spec, any_spec, any_spec],
        out_shape=[
            jax.ShapeDtypeStruct((N, H), jnp.float32),
            jax.ShapeDtypeStruct((N, H), jnp.float32),
            jax.ShapeDtypeStruct((N, H), jnp.float32),
        ],
        scratch_shapes=[
            vmem(), vmem(), vmem(),
            vmem(), vmem(), vmem(),
            pltpu.SemaphoreType.DMA((K, 3)),
            pltpu.SemaphoreType.DMA((K, 3)),
        ],
    )(x, h, c,
      W_i, Theta_i, bias_i,
      W_f, Theta_f, bias_f,
      W_c, Theta_c, bias_c,
      W_o, Theta_o, bias_o,
      W_lin, blin)
    return (hout, h0, cnew)
